# baseline (speedup 1.0000x reference)
import functools

import jax
import jax.numpy as jnp
from jax import lax
from jax.experimental import pallas as pl
from jax.experimental.pallas import tpu as pltpu
from jax.experimental.pallas import tpu_sc as plsc

_NC = 2
_NS = 16
_L = 16
_NW = _NC * _NS
_G = 8
_CH = 16


def _body(users_h, items_h, uw_h, iw_h, ub_h, ib_h, out_h,
          uidx, iidx, usub, isub, u3, i3, ub3, ib3, outv,
          sem_uw, sem_iw, sem_ub, sem_ib, *, bpw, D):
    wid = lax.axis_index("s") * _NC + lax.axis_index("c")
    base = wid * bpw

    pltpu.sync_copy(users_h.at[pl.ds(base, bpw)], uidx)
    pltpu.sync_copy(items_h.at[pl.ds(base, bpw)], iidx)
    for c in range(bpw // _L):
        s = pl.ds(c * _L, _L)
        um1 = uidx[s] - 1
        im1 = iidx[s] - 1
        uidx[s] = um1
        iidx[s] = im1
        usub[s] = lax.bitwise_and(um1, _G - 1)
        isub[s] = lax.bitwise_and(im1, _G - 1)

    col0 = lax.iota(jnp.int32, _L)
    zz = col0 * 0

    def chunk(k, carry):
        c0 = k * _CH

        ug16 = lax.shift_right_logical(uidx[pl.ds(c0, _L)], 3)
        ig16 = lax.shift_right_logical(iidx[pl.ds(c0, _L)], 3)
        for l in range(_L):
            ub0 = pl.multiple_of(ug16[l] * _G, _G)
            ib0 = pl.multiple_of(ig16[l] * _G, _G)
            pltpu.async_copy(uw_h.at[pl.ds(ub0, _G)], u3.at[l], sem_uw)
            pltpu.async_copy(iw_h.at[pl.ds(ib0, _G)], i3.at[l], sem_iw)
            pltpu.async_copy(ub_h.at[pl.ds(ub0, _G)], ub3.at[l], sem_ub)
            pltpu.async_copy(ib_h.at[pl.ds(ib0, _G)], ib3.at[l], sem_ib)

        for l in range(_L):
            pltpu.make_async_copy(uw_h.at[pl.ds(0, _G)], u3.at[l],
                                  sem_uw).wait()
            pltpu.make_async_copy(iw_h.at[pl.ds(0, _G)], i3.at[l],
                                  sem_iw).wait()
            pltpu.make_async_copy(ub_h.at[pl.ds(0, _G)], ub3.at[l],
                                  sem_ub).wait()
            pltpu.make_async_copy(ib_h.at[pl.ds(0, _G)], ib3.at[l],
                                  sem_ib).wait()

        us16 = usub[pl.ds(c0, _L)]
        is16 = isub[pl.ds(c0, _L)]
        acc = (plsc.load_gather(ub3, [col0, us16, zz])
               + plsc.load_gather(ib3, [col0, is16, zz]))
        for d in range(D):
            dd = lax.bitwise_and(col0 + d, D - 1)
            uv = plsc.load_gather(u3, [col0, us16, dd])
            iv = plsc.load_gather(i3, [col0, is16, dd])
            acc = acc + uv * iv
        outv[pl.ds(c0, _L)] = 5.0 / (1.0 + jnp.exp(-acc))
        return carry

    lax.fori_loop(0, bpw // _CH, chunk, 0)

    pltpu.sync_copy(outv, out_h.at[pl.ds(base, bpw)])


@jax.jit
def kernel(users, items, u_weight, i_weight, u_bias, i_bias):
    B = users.shape[0]
    N, D = u_weight.shape
    bpw = B // _NW
    mesh = plsc.VectorSubcoreMesh(core_axis_name="c", subcore_axis_name="s")
    f = pl.kernel(
        functools.partial(_body, bpw=bpw, D=D),
        out_type=jax.ShapeDtypeStruct((B,), jnp.float32),
        mesh=mesh,
        compiler_params=pltpu.CompilerParams(needs_layout_passes=False),
        scratch_types=[
            pltpu.VMEM((bpw,), jnp.int32),
            pltpu.VMEM((bpw,), jnp.int32),
            pltpu.VMEM((bpw,), jnp.int32),
            pltpu.VMEM((bpw,), jnp.int32),
            pltpu.VMEM((_CH, _G, D), jnp.float32),
            pltpu.VMEM((_CH, _G, D), jnp.float32),
            pltpu.VMEM((_CH, _G, 1), jnp.float32),
            pltpu.VMEM((_CH, _G, 1), jnp.float32),
            pltpu.VMEM((bpw,), jnp.float32),
            pltpu.SemaphoreType.DMA,
            pltpu.SemaphoreType.DMA,
            pltpu.SemaphoreType.DMA,
            pltpu.SemaphoreType.DMA,
        ],
    )
    return f(users, items, u_weight, i_weight, u_bias, i_bias)


# speedup vs baseline: 1.0108x; 1.0108x over previous
import functools

import jax
import jax.numpy as jnp
from jax import lax
from jax.experimental import pallas as pl
from jax.experimental.pallas import tpu as pltpu
from jax.experimental.pallas import tpu_sc as plsc

_NC = 2
_NS = 16
_L = 16
_NW = _NC * _NS
_G = 8
_CH = 16
_NBUF = 2


def _body(users_h, items_h, uw_h, iw_h, ub_h, ib_h, out_h,
          uidx, iidx, u3, i3, ub3, ib3, outv,
          sem_uw, sem_iw, sem_ub, sem_ib, *, bpw, D):
    wid = lax.axis_index("s") * _NC + lax.axis_index("c")
    base = wid * bpw
    nch = bpw // _CH

    pltpu.sync_copy(users_h.at[pl.ds(base, bpw)], uidx)
    pltpu.sync_copy(items_h.at[pl.ds(base, bpw)], iidx)
    for c in range(bpw // _L):
        s = pl.ds(c * _L, _L)
        uidx[s] = uidx[s] - 1
        iidx[s] = iidx[s] - 1

    col0 = lax.iota(jnp.int32, _L)
    zz = col0 * 0

    def fire_weights(k, s):
        ug16 = lax.shift_right_logical(uidx[pl.ds(k * _CH, _L)], 3)
        ig16 = lax.shift_right_logical(iidx[pl.ds(k * _CH, _L)], 3)
        for l in range(_L):
            ub0 = pl.multiple_of(ug16[l] * _G, _G)
            ib0 = pl.multiple_of(ig16[l] * _G, _G)
            pltpu.async_copy(uw_h.at[pl.ds(ub0, _G)],
                             u3.at[s * _CH + l], sem_uw)
            pltpu.async_copy(iw_h.at[pl.ds(ib0, _G)],
                             i3.at[s * _CH + l], sem_iw)

    for s in range(_NBUF):
        fire_weights(s, s)

    def pair(m, carry):
        for s in range(_NBUF):
            k = _NBUF * m + s
            c0 = k * _CH
            uraw16 = uidx[pl.ds(c0, _L)]
            iraw16 = iidx[pl.ds(c0, _L)]
            ug16 = lax.shift_right_logical(uraw16, 3)
            ig16 = lax.shift_right_logical(iraw16, 3)

            for l in range(_L):
                ub0 = pl.multiple_of(ug16[l] * _G, _G)
                ib0 = pl.multiple_of(ig16[l] * _G, _G)
                pltpu.async_copy(ub_h.at[pl.ds(ub0, _G)], ub3.at[l], sem_ub)
                pltpu.async_copy(ib_h.at[pl.ds(ib0, _G)], ib3.at[l], sem_ib)

            for l in range(_L):
                pltpu.make_async_copy(uw_h.at[pl.ds(0, _G)],
                                      u3.at[s * _CH + l], sem_uw).wait()
                pltpu.make_async_copy(iw_h.at[pl.ds(0, _G)],
                                      i3.at[s * _CH + l], sem_iw).wait()

            @pl.when(k + _NBUF < nch)
            def _():
                fire_weights(k + _NBUF, s)

            for l in range(_L):
                pltpu.make_async_copy(ub_h.at[pl.ds(0, _G)], ub3.at[l],
                                      sem_ub).wait()
                pltpu.make_async_copy(ib_h.at[pl.ds(0, _G)], ib3.at[l],
                                      sem_ib).wait()

            us16 = lax.bitwise_and(uraw16, _G - 1)
            is16 = lax.bitwise_and(iraw16, _G - 1)
            acc = (plsc.load_gather(ub3, [col0, us16, zz])
                   + plsc.load_gather(ib3, [col0, is16, zz]))
            row16 = col0 + s * _CH
            for d in range(D):
                dd = lax.bitwise_and(col0 + d, D - 1)
                uv = plsc.load_gather(u3, [row16, us16, dd])
                iv = plsc.load_gather(i3, [row16, is16, dd])
                acc = acc + uv * iv
            outv[pl.ds(c0, _L)] = 5.0 / (1.0 + jnp.exp(-acc))
        return carry

    lax.fori_loop(0, nch // _NBUF, pair, 0)

    pltpu.sync_copy(outv, out_h.at[pl.ds(base, bpw)])


@jax.jit
def kernel(users, items, u_weight, i_weight, u_bias, i_bias):
    B = users.shape[0]
    N, D = u_weight.shape
    bpw = B // _NW
    mesh = plsc.VectorSubcoreMesh(core_axis_name="c", subcore_axis_name="s")
    f = pl.kernel(
        functools.partial(_body, bpw=bpw, D=D),
        out_type=jax.ShapeDtypeStruct((B,), jnp.float32),
        mesh=mesh,
        compiler_params=pltpu.CompilerParams(needs_layout_passes=False),
        scratch_types=[
            pltpu.VMEM((bpw,), jnp.int32),
            pltpu.VMEM((bpw,), jnp.int32),
            pltpu.VMEM((_NBUF * _CH, _G, D), jnp.float32),
            pltpu.VMEM((_NBUF * _CH, _G, D), jnp.float32),
            pltpu.VMEM((_CH, _G, 1), jnp.float32),
            pltpu.VMEM((_CH, _G, 1), jnp.float32),
            pltpu.VMEM((bpw,), jnp.float32),
            pltpu.SemaphoreType.DMA,
            pltpu.SemaphoreType.DMA,
            pltpu.SemaphoreType.DMA,
            pltpu.SemaphoreType.DMA,
        ],
    )
    return f(users, items, u_weight, i_weight, u_bias, i_bias)


# speedup vs baseline: 1.4739x; 1.4582x over previous
import functools

import jax
import jax.numpy as jnp
from jax import lax
from jax.experimental import pallas as pl
from jax.experimental.pallas import tpu as pltpu
from jax.experimental.pallas import tpu_sc as plsc

_NC = 2
_NS = 16
_L = 16
_NW = _NC * _NS
_G = 8
_CH = 32


def _body(users_h, items_h, uw_h, iw_h, ub_h, ib_h, out_h,
          uidx, iidx, usub, isub, u3, i3, ubv, ibv, outv,
          sem_uw, sem_iw, sem_ub, sem_ib, *, bpw, D):
    wid = lax.axis_index("s") * _NC + lax.axis_index("c")
    base = wid * bpw

    pltpu.sync_copy(users_h.at[pl.ds(base, bpw)], uidx)
    pltpu.sync_copy(items_h.at[pl.ds(base, bpw)], iidx)
    for c in range(bpw // _L):
        s = pl.ds(c * _L, _L)
        um1 = uidx[s] - 1
        im1 = iidx[s] - 1
        uidx[s] = um1
        iidx[s] = im1
        usub[s] = lax.bitwise_and(um1, _G - 1)
        isub[s] = lax.bitwise_and(im1, _G - 1)

    cp_ub = pltpu.async_copy(ub_h.at[uidx], ubv, sem_ub)
    cp_ib = pltpu.async_copy(ib_h.at[iidx], ibv, sem_ib)
    cp_ub.wait()
    cp_ib.wait()

    col0 = lax.iota(jnp.int32, _L)

    def chunk(k, carry):
        c0 = k * _CH

        for q in range(_CH // _L):
            ug16 = lax.shift_right_logical(
                uidx[pl.ds(c0 + q * _L, _L)], 3)
            ig16 = lax.shift_right_logical(
                iidx[pl.ds(c0 + q * _L, _L)], 3)
            for l in range(_L):
                jj = q * _L + l
                pltpu.async_copy(
                    uw_h.at[pl.ds(ug16[l] * _G, _G)], u3.at[jj], sem_uw)
                pltpu.async_copy(
                    iw_h.at[pl.ds(ig16[l] * _G, _G)], i3.at[jj], sem_iw)

        for q in range(_CH // _L):
            ug16 = lax.shift_right_logical(
                uidx[pl.ds(c0 + q * _L, _L)], 3)
            ig16 = lax.shift_right_logical(
                iidx[pl.ds(c0 + q * _L, _L)], 3)
            for l in range(_L):
                jj = q * _L + l
                pltpu.make_async_copy(
                    uw_h.at[pl.ds(ug16[l] * _G, _G)], u3.at[jj],
                    sem_uw).wait()
                pltpu.make_async_copy(
                    iw_h.at[pl.ds(ig16[l] * _G, _G)], i3.at[jj],
                    sem_iw).wait()

        for g in range(_CH // _L):
            b0 = c0 + g * _L
            row16 = col0 + g * _L
            us16 = usub[pl.ds(b0, _L)]
            is16 = isub[pl.ds(b0, _L)]
            acc = ubv[pl.ds(b0, _L)] + ibv[pl.ds(b0, _L)]
            for d in range(D):
                dd = lax.bitwise_and(col0 + d, D - 1)
                uv = plsc.load_gather(u3, [row16, us16, dd])
                iv = plsc.load_gather(i3, [row16, is16, dd])
                acc = acc + uv * iv
            outv[pl.ds(b0, _L)] = 5.0 / (1.0 + jnp.exp(-acc))
        return carry

    lax.fori_loop(0, bpw // _CH, chunk, 0)

    pltpu.sync_copy(outv, out_h.at[pl.ds(base, bpw)])


@jax.jit
def kernel(users, items, u_weight, i_weight, u_bias, i_bias):
    B = users.shape[0]
    N, D = u_weight.shape
    bpw = B // _NW
    mesh = plsc.VectorSubcoreMesh(core_axis_name="c", subcore_axis_name="s")
    f = pl.kernel(
        functools.partial(_body, bpw=bpw, D=D),
        out_type=jax.ShapeDtypeStruct((B,), jnp.float32),
        mesh=mesh,
        compiler_params=pltpu.CompilerParams(needs_layout_passes=False),
        scratch_types=[
            pltpu.VMEM((bpw,), jnp.int32),
            pltpu.VMEM((bpw,), jnp.int32),
            pltpu.VMEM((bpw,), jnp.int32),
            pltpu.VMEM((bpw,), jnp.int32),
            pltpu.VMEM((_CH, _G, D), jnp.float32),
            pltpu.VMEM((_CH, _G, D), jnp.float32),
            pltpu.VMEM((bpw,), jnp.float32),
            pltpu.VMEM((bpw,), jnp.float32),
            pltpu.VMEM((bpw,), jnp.float32),
            pltpu.SemaphoreType.DMA,
            pltpu.SemaphoreType.DMA,
            pltpu.SemaphoreType.DMA,
            pltpu.SemaphoreType.DMA,
        ],
    )
    return f(users, items, u_weight, i_weight,
             u_bias.reshape(-1), i_bias.reshape(-1))
